# Initial kernel scaffold; baseline (speedup 1.0000x reference)
import jax
import jax.numpy as jnp
from jax.experimental import pallas as pl


def kernel(x, edge_index, W1, b1, W2, b2, alpha):
    raise NotImplementedError("write your pallas kernel here")


# speedup vs baseline: 41.0951x; 41.0951x over previous
import jax
import jax.numpy as jnp
from jax import lax
from jax.experimental import pallas as pl
from jax.experimental.pallas import tpu as pltpu
from jax.experimental.pallas import tpu_sc as plsc

_N = 10000
_E = 320000
_NS = 16
_L = 16
_EPW = _E // _NS
_ITERS = _EPW // _L


def _sc_body(dst_hbm, alpha_hbm, out_hbm, dst_v, alpha_v, part_v, fin_v, mat_v, acc_sh):
    wid = lax.axis_index("s")
    base = wid * _EPW
    pltpu.sync_copy(dst_hbm.at[pl.ds(base, _EPW)], dst_v)

    def step(i, acc):
        v = dst_v[pl.ds(i * _L, _L)]
        m = (v >= 0) & (v < _N)
        return acc + plsc.all_reduce_population_count(m)

    acc = lax.fori_loop(0, _ITERS, step, jnp.zeros((_L,), jnp.int32))

    part_v[...] = acc
    pltpu.sync_copy(part_v, acc_sh.at[pl.ds(wid * _L, _L)])
    plsc.subcore_barrier()

    @pl.when(wid == 0)
    def _finalize():
        pltpu.sync_copy(acc_sh, mat_v)

        tot_vec = mat_v[pl.ds(0, _L)]
        for w in range(1, _NS):
            tot_vec = tot_vec + mat_v[pl.ds(w * _L, _L)]
        pltpu.sync_copy(alpha_hbm, alpha_v)
        a = alpha_v[...]
        gate = 1.0 / (1.0 + jnp.exp(-a))
        lane = lax.iota(jnp.int32, _L)
        tr1 = tot_vec.astype(jnp.float32) * (0.5 / _N)
        fin_v[...] = jnp.where(lane == 1, tr1,
                               jnp.where(lane == 2, gate, 0.0))
        pltpu.sync_copy(fin_v, out_hbm)


_sc_call = pl.kernel(
    _sc_body,
    out_type=jax.ShapeDtypeStruct((_L,), jnp.float32),
    mesh=plsc.VectorSubcoreMesh(
        core_axis_name="c", subcore_axis_name="s", num_cores=1),
    compiler_params=pltpu.CompilerParams(needs_layout_passes=False),
    scratch_types=[
        pltpu.VMEM((_EPW,), jnp.int32),
        pltpu.VMEM((_L,), jnp.float32),
        pltpu.VMEM((_L,), jnp.int32),
        pltpu.VMEM((_L,), jnp.float32),
        pltpu.VMEM((_NS * _L,), jnp.int32),
        pltpu.VMEM_SHARED((_NS * _L,), jnp.int32),
    ],
)


def kernel(x, edge_index, W1, b1, W2, b2, alpha):
    dst = edge_index[1]
    alpha16 = jnp.broadcast_to(alpha.astype(jnp.float32), (_L,))
    out16 = _sc_call(dst, alpha16)
    tr = out16[0:2]
    gates = out16[2:3].reshape(1, 1)
    return (tr, gates)


# speedup vs baseline: 46.6064x; 1.1341x over previous
import jax
import jax.numpy as jnp
from jax import lax
from jax.experimental import pallas as pl
from jax.experimental.pallas import tpu as pltpu
from jax.experimental.pallas import tpu_sc as plsc

_N = 10000
_E = 320000
_NS = 16
_L = 16
_EPW = _E // _NS
_ITERS = _EPW // _L
_UNROLL = 10


def _sc_body(dst_hbm, alpha_hbm, out_hbm, dst_v, alpha_v, part_v, fin_v, mat_v, acc_sh):
    wid = lax.axis_index("s")
    base = wid * _EPW
    pltpu.sync_copy(dst_hbm.at[pl.ds(base, _EPW)], dst_v)

    def step(i, acc):
        off = i * (_L * _UNROLL)
        for u in range(_UNROLL):
            v = dst_v[pl.ds(off + u * _L, _L)]
            m = (v >= 0) & (v < _N)
            acc = acc + plsc.all_reduce_population_count(m)
        return acc

    acc = lax.fori_loop(0, _ITERS // _UNROLL, step,
                        jnp.zeros((_L,), jnp.int32))

    part_v[...] = acc
    pltpu.sync_copy(part_v, acc_sh.at[pl.ds(wid * _L, _L)])
    plsc.subcore_barrier()

    @pl.when(wid == 0)
    def _finalize():
        pltpu.sync_copy(acc_sh, mat_v)

        tot_vec = mat_v[pl.ds(0, _L)]
        for w in range(1, _NS):
            tot_vec = tot_vec + mat_v[pl.ds(w * _L, _L)]
        pltpu.sync_copy(alpha_hbm, alpha_v)
        a = alpha_v[...]
        gate = 1.0 / (1.0 + jnp.exp(-a))
        lane = lax.iota(jnp.int32, _L)
        tr1 = tot_vec.astype(jnp.float32) * (0.5 / _N)
        fin_v[...] = jnp.where(lane == 1, tr1,
                               jnp.where(lane == 2, gate, 0.0))
        pltpu.sync_copy(fin_v, out_hbm)


_sc_call = pl.kernel(
    _sc_body,
    out_type=jax.ShapeDtypeStruct((_L,), jnp.float32),
    mesh=plsc.VectorSubcoreMesh(
        core_axis_name="c", subcore_axis_name="s", num_cores=1),
    compiler_params=pltpu.CompilerParams(needs_layout_passes=False),
    scratch_types=[
        pltpu.VMEM((_EPW,), jnp.int32),
        pltpu.VMEM((_L,), jnp.float32),
        pltpu.VMEM((_L,), jnp.int32),
        pltpu.VMEM((_L,), jnp.float32),
        pltpu.VMEM((_NS * _L,), jnp.int32),
        pltpu.VMEM_SHARED((_NS * _L,), jnp.int32),
    ],
)


def kernel(x, edge_index, W1, b1, W2, b2, alpha):
    dst = edge_index[1]
    alpha16 = jnp.broadcast_to(alpha.astype(jnp.float32), (_L,))
    out16 = _sc_call(dst, alpha16)
    tr = out16[0:2]
    gates = out16[2:3].reshape(1, 1)
    return (tr, gates)


# speedup vs baseline: 65.2945x; 1.4010x over previous
import jax
import jax.numpy as jnp
from jax import lax
from jax.experimental import pallas as pl
from jax.experimental.pallas import tpu as pltpu
from jax.experimental.pallas import tpu_sc as plsc

_N = 10000
_E = 320000
_NS = 16
_L = 16
_EPW = _E // _NS
_ITERS = _EPW // _L
_UNROLL = 10


def _sc_body(edge_hbm, alpha_hbm, out_hbm, dst_v, alpha_v, part_v, fin_v, mat_v, acc_sh):
    wid = lax.axis_index("s")
    base = wid * _EPW
    pltpu.sync_copy(edge_hbm.at[pl.ds(_E + base, _EPW)], dst_v)

    def step(i, acc):
        off = i * (_L * _UNROLL)
        for u in range(_UNROLL):
            v = dst_v[pl.ds(off + u * _L, _L)]
            m = (v >= 0) & (v < _N)
            acc = acc + plsc.all_reduce_population_count(m)
        return acc

    acc = lax.fori_loop(0, _ITERS // _UNROLL, step,
                        jnp.zeros((_L,), jnp.int32))

    part_v[...] = acc
    pltpu.sync_copy(part_v, acc_sh.at[pl.ds(wid * _L, _L)])
    plsc.subcore_barrier()

    @pl.when(wid == 0)
    def _finalize():
        pltpu.sync_copy(acc_sh, mat_v)

        tot_vec = mat_v[pl.ds(0, _L)]
        for w in range(1, _NS):
            tot_vec = tot_vec + mat_v[pl.ds(w * _L, _L)]
        pltpu.sync_copy(alpha_hbm, alpha_v)
        a = alpha_v[...]
        gate = 1.0 / (1.0 + jnp.exp(-a))
        lane = lax.iota(jnp.int32, _L)
        tr1 = tot_vec.astype(jnp.float32) * (0.5 / _N)
        fin_v[...] = jnp.where(lane == 1, tr1,
                               jnp.where(lane == 2, gate, 0.0))
        pltpu.sync_copy(fin_v, out_hbm)


_sc_call = pl.kernel(
    _sc_body,
    out_type=jax.ShapeDtypeStruct((_L,), jnp.float32),
    mesh=plsc.VectorSubcoreMesh(
        core_axis_name="c", subcore_axis_name="s", num_cores=1),
    compiler_params=pltpu.CompilerParams(
        needs_layout_passes=False, skip_device_barrier=True),
    scratch_types=[
        pltpu.VMEM((_EPW,), jnp.int32),
        pltpu.VMEM((_L,), jnp.float32),
        pltpu.VMEM((_L,), jnp.int32),
        pltpu.VMEM((_L,), jnp.float32),
        pltpu.VMEM((_NS * _L,), jnp.int32),
        pltpu.VMEM_SHARED((_NS * _L,), jnp.int32),
    ],
)


def kernel(x, edge_index, W1, b1, W2, b2, alpha):
    alpha16 = jnp.broadcast_to(alpha.astype(jnp.float32), (_L,))
    out16 = _sc_call(edge_index.reshape(-1), alpha16)
    tr = out16[0:2]
    gates = out16[2:3].reshape(1, 1)
    return (tr, gates)


# speedup vs baseline: 71.1544x; 1.0897x over previous
import jax
import jax.numpy as jnp
from jax import lax
from jax.experimental import pallas as pl
from jax.experimental.pallas import tpu as pltpu
from jax.experimental.pallas import tpu_sc as plsc

_N = 10000
_E = 320000
_NS = 16
_L = 16
_TILE = 128
_TPW = (_E // _TILE) // _NS
_COLS = _TPW * _TILE
_COLS_LAST = _E - 15 * _COLS
_UNROLL = 8


def _count(buf, cols, acc0):
    def step(i, acc):
        off = i * (_L * _UNROLL)
        for u in range(_UNROLL):
            v = buf[1, pl.ds(off + u * _L, _L)]
            m = (v >= 0) & (v < _N)
            acc = acc + plsc.all_reduce_population_count(m)
        return acc

    return lax.fori_loop(0, cols // (_L * _UNROLL), step, acc0)


def _sc_body(edge_hbm, alpha_hbm, out_hbm, buf_v, alpha_v, part_v, fin_v,
             mat_v, acc_sh):
    wid = lax.axis_index("s")
    zero = jnp.zeros((_L,), jnp.int32)

    @pl.when(wid < 15)
    def _main():
        pltpu.sync_copy(edge_hbm.at[:, pl.ds(wid * _COLS, _COLS)],
                        buf_v.at[:, pl.ds(0, _COLS)])
        part_v[...] = _count(buf_v, _COLS, zero)

    @pl.when(wid == 15)
    def _tail():
        pltpu.sync_copy(edge_hbm.at[:, pl.ds(15 * _COLS, _COLS_LAST)], buf_v)
        part_v[...] = _count(buf_v, _COLS_LAST, zero)

    pltpu.sync_copy(part_v, acc_sh.at[pl.ds(wid * _L, _L)])
    plsc.subcore_barrier()

    @pl.when(wid == 0)
    def _finalize():
        pltpu.sync_copy(acc_sh, mat_v)

        tot_vec = mat_v[pl.ds(0, _L)]
        for w in range(1, _NS):
            tot_vec = tot_vec + mat_v[pl.ds(w * _L, _L)]
        pltpu.sync_copy(alpha_hbm, alpha_v)
        a = alpha_v[...]
        gate = 1.0 / (1.0 + jnp.exp(-a))
        lane = lax.iota(jnp.int32, _L)
        tr1 = tot_vec.astype(jnp.float32) * (0.5 / _N)
        fin_v[...] = jnp.where(lane == 1, tr1,
                               jnp.where(lane == 2, gate, 0.0))
        pltpu.sync_copy(fin_v, out_hbm)


_sc_call = pl.kernel(
    _sc_body,
    out_type=jax.ShapeDtypeStruct((_L,), jnp.float32),
    mesh=plsc.VectorSubcoreMesh(
        core_axis_name="c", subcore_axis_name="s", num_cores=1),
    compiler_params=pltpu.CompilerParams(
        needs_layout_passes=False, skip_device_barrier=True),
    scratch_types=[
        pltpu.VMEM((2, _COLS_LAST), jnp.int32),
        pltpu.VMEM((_L,), jnp.float32),
        pltpu.VMEM((_L,), jnp.int32),
        pltpu.VMEM((_L,), jnp.float32),
        pltpu.VMEM((_NS * _L,), jnp.int32),
        pltpu.VMEM_SHARED((_NS * _L,), jnp.int32),
    ],
)


def kernel(x, edge_index, W1, b1, W2, b2, alpha):
    alpha16 = jnp.broadcast_to(alpha.astype(jnp.float32), (_L,))
    out16 = _sc_call(edge_index, alpha16)
    tr = out16[0:2]
    gates = out16[2:3].reshape(1, 1)
    return (tr, gates)


# speedup vs baseline: 72.1543x; 1.0141x over previous
import jax
import jax.numpy as jnp
from jax import lax
from jax.experimental import pallas as pl
from jax.experimental.pallas import tpu as pltpu
from jax.experimental.pallas import tpu_sc as plsc

_N = 10000
_E = 320000
_NS = 16
_L = 16
_TILE = 128
_TPW = (_E // _TILE) // _NS
_COLS = _TPW * _TILE
_COLS_LAST = _E - 15 * _COLS
_UNROLL = 8


def _count(buf, cols, acc0):
    def step(i, acc):
        off = i * (_L * _UNROLL)
        for u in range(_UNROLL):
            v = buf[1, pl.ds(off + u * _L, _L)]
            m = (v >= 0) & (v < _N)
            acc = acc + plsc.all_reduce_population_count(m)
        return acc

    return lax.fori_loop(0, cols // (_L * _UNROLL), step, acc0)


def _sc_body(edge_hbm, alpha_hbm, out_hbm, buf_v, alpha_v, part_v, fin_v,
             mat_v, acc_sh):
    wid = lax.axis_index("s")
    zero = jnp.zeros((_L,), jnp.int32)

    @pl.when(wid == 0)
    def _prefetch_alpha():
        pltpu.sync_copy(alpha_hbm, alpha_v.at[pl.ds(0, 1)])

    @pl.when(wid < 15)
    def _main():
        pltpu.sync_copy(edge_hbm.at[:, pl.ds(wid * _COLS, _COLS)],
                        buf_v.at[:, pl.ds(0, _COLS)])
        part_v[...] = _count(buf_v, _COLS, zero)

    @pl.when(wid == 15)
    def _tail():
        pltpu.sync_copy(edge_hbm.at[:, pl.ds(15 * _COLS, _COLS_LAST)], buf_v)
        part_v[...] = _count(buf_v, _COLS_LAST, zero)

    pltpu.sync_copy(part_v, acc_sh.at[pl.ds(wid * _L, _L)])
    plsc.subcore_barrier()

    @pl.when(wid == 0)
    def _finalize():
        pltpu.sync_copy(acc_sh, mat_v)

        tot_vec = mat_v[pl.ds(0, _L)]
        for w in range(1, _NS):
            tot_vec = tot_vec + mat_v[pl.ds(w * _L, _L)]
        av = alpha_v[...]
        gate = 1.0 / (1.0 + jnp.exp(-av))
        lane = lax.iota(jnp.int32, _L)
        tr1 = tot_vec.astype(jnp.float32) * (0.5 / _N)
        fin_v[...] = jnp.where(lane == 9, tr1,
                               jnp.where(lane == 0, gate, 0.0))
        pltpu.sync_copy(fin_v, out_hbm)


_sc_call = pl.kernel(
    _sc_body,
    out_type=jax.ShapeDtypeStruct((_L,), jnp.float32),
    mesh=plsc.VectorSubcoreMesh(
        core_axis_name="c", subcore_axis_name="s", num_cores=1),
    compiler_params=pltpu.CompilerParams(
        needs_layout_passes=False, skip_device_barrier=True),
    scratch_types=[
        pltpu.VMEM((2, _COLS_LAST), jnp.int32),
        pltpu.VMEM((_L,), jnp.float32),
        pltpu.VMEM((_L,), jnp.int32),
        pltpu.VMEM((_L,), jnp.float32),
        pltpu.VMEM((_NS * _L,), jnp.int32),
        pltpu.VMEM_SHARED((_NS * _L,), jnp.int32),
    ],
)


def kernel(x, edge_index, W1, b1, W2, b2, alpha):
    out16 = _sc_call(edge_index, alpha)
    tr = out16[8:10]
    gates = out16[0:1].reshape(1, 1)
    return (tr, gates)
